# baseline (speedup 1.0000x reference)
import functools

import jax
import jax.numpy as jnp
from jax import lax
from jax.experimental import pallas as pl
from jax.experimental.pallas import tpu as pltpu
from jax.experimental.pallas import tpu_sc as plsc

VOCAB = 1000000
DIM = 64
BATCH = 16384
CTX = 20

NC = 2
NS = 16
NW = NC * NS
BPW = BATCH // NW
STEP = 64
NSTEP = BPW // STEP
NK = DIM // 16


def _body(cs_hbm, os_hbm, word_hbm, bkp_hbm, out_hbm,
          idx_os, idx_cs, bufs, cbuf, prow, ysig, sem):
    wid = lax.axis_index("s") * NC + lax.axis_index("c")
    base = wid * BPW

    pltpu.sync_copy(cs_hbm.at[pl.ds(base, BPW)], idx_cs)
    for c in range(CTX):
        pltpu.sync_copy(os_hbm.at[c, pl.ds(base, BPW)], idx_os.at[c])

    lane = lax.iota(jnp.int32, 16)

    def step(si, carry):
        sbase = si * STEP
        copies = []
        for c in range(CTX):
            cp = pltpu.make_async_copy(
                word_hbm.at[idx_os.at[c, pl.ds(sbase, STEP)]], bufs.at[c], sem)
            cp.start()
            copies.append(cp)
        cpc = pltpu.make_async_copy(
            bkp_hbm.at[idx_cs.at[pl.ds(sbase, STEP)]], cbuf, sem)
        cpc.start()
        for cp in copies:
            cp.wait()
        cpc.wait()

        def row(r, rcarry):
            pr = jnp.zeros((16,), jnp.float32)
            for k in range(NK):
                a = bufs[0, r, pl.ds(k * 16, 16)]
                for c in range(1, CTX):
                    a = a + bufs[c, r, pl.ds(k * 16, 16)]
                pr = pr + a * cbuf[r, pl.ds(k * 16, 16)]
            prow[r] = pr * (1.0 / CTX)
            return rcarry

        lax.fori_loop(0, STEP, row, 0, unroll=2)

        for g in range(STEP // 16):
            y = jnp.zeros((16,), jnp.float32)
            for l in range(16):
                s = jnp.sum(prow[g * 16 + l])
                y = jnp.where(lane == l, s, y)
            ysig[pl.ds(sbase + g * 16, 16)] = 1.0 / (1.0 + jnp.exp(-y))
        return carry

    lax.fori_loop(0, NSTEP, step, 0)

    pltpu.sync_copy(ysig, out_hbm.at[pl.ds(base, BPW)])


@functools.partial(jax.jit, static_argnames=())
def _cbow(cs, os, word_embs, bkp_word_embs):
    mesh = plsc.VectorSubcoreMesh(core_axis_name="c", subcore_axis_name="s")
    f = pl.kernel(
        _body,
        out_type=jax.ShapeDtypeStruct((BATCH,), jnp.float32),
        mesh=mesh,
        compiler_params=pltpu.CompilerParams(
            needs_layout_passes=False, use_tc_tiling_on_sc=False),
        scratch_types=[
            pltpu.VMEM((CTX, BPW), jnp.int32),
            pltpu.VMEM((BPW,), jnp.int32),
            pltpu.VMEM((CTX, STEP, DIM), jnp.float32),
            pltpu.VMEM((STEP, DIM), jnp.float32),
            pltpu.VMEM((STEP, 16), jnp.float32),
            pltpu.VMEM((BPW,), jnp.float32),
            pltpu.SemaphoreType.DMA,
        ],
    )
    return f(cs, os, word_embs, bkp_word_embs)


def kernel(cs, os, word_embs, bkp_word_embs):
    return _cbow(cs, os, word_embs, bkp_word_embs)
